# baseline (speedup 1.0000x reference)
import functools

import jax
import jax.numpy as jnp
from jax import lax
from jax.experimental import pallas as pl
from jax.experimental.pallas import tpu as pltpu
from jax.experimental.pallas import tpu_sc as plsc

HIDDEN = 1024
ROWS = 4
COLS = 8192
NC = 2
NS = 16
NW = NC * NS
B = ROWS * COLS
B_PER_W = B // NW
W_PER_ROW = COLS // B_PER_W
CHUNK = 16
N_CHUNKS = B_PER_W // CHUNK
NBUF = 4
N_STEPS = N_CHUNKS // NBUF


def _gather_body(ids_hbm, table_hbm, out_hbm, idx_v, bufs, gsems, osems):
    wid = lax.axis_index("s") * NC + lax.axis_index("c")
    r = wid // W_PER_ROW
    cb = pl.multiple_of((wid % W_PER_ROW) * B_PER_W, B_PER_W)
    pltpu.sync_copy(ids_hbm.at[r, pl.ds(cb, B_PER_W)], idx_v)

    def gather_desc(c, b):
        off = pl.multiple_of(c * CHUNK, CHUNK)
        return pltpu.make_async_copy(
            table_hbm.at[idx_v.at[pl.ds(off, CHUNK)]], bufs[b], gsems[b])

    def out_desc(c, b):
        off = pl.multiple_of(c * CHUNK, CHUNK)
        return pltpu.make_async_copy(
            bufs[b], out_hbm.at[r, pl.ds(cb + off, CHUNK), :], osems[b])

    gather_desc(0, 0).start()
    gather_desc(1, 1).start()

    def step(t, carry):
        c0 = t * NBUF
        for j in range(NBUF):
            c = c0 + j
            gather_desc(c, j).wait()

            @pl.when(c >= 2)
            def _():
                out_desc(c - 2, (j + 2) % NBUF).wait()

            @pl.when(c + 2 < N_CHUNKS)
            def _():
                gather_desc(c + 2, (j + 2) % NBUF).start()

            out_desc(c, j).start()
        return carry

    lax.fori_loop(0, N_STEPS, step, 0)
    out_desc(N_CHUNKS - 2, 2).wait()
    out_desc(N_CHUNKS - 1, 3).wait()


@functools.partial(
    pl.kernel,
    out_type=jax.ShapeDtypeStruct((ROWS, COLS, HIDDEN), jnp.float32),
    mesh=plsc.VectorSubcoreMesh(core_axis_name="c", subcore_axis_name="s"),
    scratch_types=(
        [pltpu.VMEM((B_PER_W,), jnp.int32)]
        + [pltpu.VMEM((CHUNK, HIDDEN), jnp.float32) for _ in range(NBUF)]
        + [pltpu.SemaphoreType.DMA] * (2 * NBUF)
    ),
)
def _sc_gather(ids_hbm, table_hbm, out_hbm, idx_v, b0, b1, b2, b3,
               g0, g1, g2, g3, o0, o1, o2, o3):
    _gather_body(ids_hbm, table_hbm, out_hbm, idx_v,
                 [b0, b1, b2, b3], [g0, g1, g2, g3], [o0, o1, o2, o3])


@jax.jit
def kernel(position_ids, table):
    return _sc_gather(position_ids, table)


# speedup vs baseline: 1.0011x; 1.0011x over previous
import functools

import jax
import jax.numpy as jnp
from jax import lax
from jax.experimental import pallas as pl
from jax.experimental.pallas import tpu as pltpu
from jax.experimental.pallas import tpu_sc as plsc

HIDDEN = 1024
NC = 2
NS = 16
NW = NC * NS
B = 4 * 8192
B_PER_W = B // NW
CHUNK = 8
N_CHUNKS = B_PER_W // CHUNK
NBUF = 8
LOOK = NBUF // 2
N_STEPS = N_CHUNKS // NBUF


def _gather_body(ids_hbm, table_hbm, out_hbm, idx_v, bufs, gsems, osems):
    wid = lax.axis_index("s") * NC + lax.axis_index("c")
    base = pl.multiple_of(wid * B_PER_W, B_PER_W)
    pltpu.sync_copy(ids_hbm.at[pl.ds(base, B_PER_W)], idx_v)

    def gather_desc(c, b):
        off = pl.multiple_of(c * CHUNK, CHUNK)
        return pltpu.make_async_copy(
            table_hbm.at[idx_v.at[pl.ds(off, CHUNK)]], bufs[b], gsems[b])

    def out_desc(c, b):
        off = pl.multiple_of(c * CHUNK, CHUNK)
        return pltpu.make_async_copy(
            bufs[b], out_hbm.at[pl.ds(base + off, CHUNK)], osems[b])

    for j in range(LOOK):
        gather_desc(j, j).start()

    def step(t, carry):
        c0 = t * NBUF
        for j in range(NBUF):
            c = c0 + j
            b2 = (j + LOOK) % NBUF
            gather_desc(c, j).wait()

            @pl.when(c >= NBUF - LOOK)
            def _():
                out_desc(c - (NBUF - LOOK), b2).wait()

            @pl.when(c + LOOK < N_CHUNKS)
            def _():
                gather_desc(c + LOOK, b2).start()

            out_desc(c, j).start()
        return carry

    lax.fori_loop(0, N_STEPS, step, 0)
    for k in range(NBUF - LOOK):
        c = N_CHUNKS - (NBUF - LOOK) + k
        out_desc(c, c % NBUF).wait()


@functools.partial(
    pl.kernel,
    out_type=jax.ShapeDtypeStruct((B, HIDDEN), jnp.float32),
    mesh=plsc.VectorSubcoreMesh(core_axis_name="c", subcore_axis_name="s"),
    scratch_types=(
        [pltpu.VMEM((B_PER_W,), jnp.int32)]
        + [pltpu.VMEM((CHUNK, HIDDEN), jnp.float32) for _ in range(NBUF)]
        + [pltpu.SemaphoreType.DMA] * (2 * NBUF)
    ),
)
def _sc_gather(ids_hbm, table_hbm, out_hbm, idx_v, *scratch):
    bufs = list(scratch[:NBUF])
    gsems = list(scratch[NBUF:2 * NBUF])
    osems = list(scratch[2 * NBUF:])
    _gather_body(ids_hbm, table_hbm, out_hbm, idx_v, bufs, gsems, osems)


@jax.jit
def kernel(position_ids, table):
    ids_flat = position_ids.reshape(-1)
    out = _sc_gather(ids_flat, table)
    return out.reshape(position_ids.shape[0], position_ids.shape[1], HIDDEN)
